# baseline (speedup 1.0000x reference)
import dataclasses

import jax
import jax.numpy as jnp
from jax.experimental import pallas as pl
from jax.experimental.pallas import tpu as pltpu
from jax.experimental.pallas import tpu_sc as plsc

N_NODES = 100000
N_EDGES = 1600000
D_MODEL = 32
ROLE_VOCAB = 1000

W_NODE = 400
W_EDGE = 800


def kernel(nodes, edges, edge_index, concept_table, role_table):
    del edge_index

    nodes2d = nodes.reshape(N_NODES // W_NODE, W_NODE).astype(jnp.int32)
    edges2d = edges.reshape(N_EDGES // W_EDGE, W_EDGE).astype(jnp.int32)

    mesh = plsc.VectorSubcoreMesh(core_axis_name="core",
                                  subcore_axis_name="subcore")

    @pl.kernel(
        out_type=jax.ShapeDtypeStruct((N_NODES + N_EDGES, D_MODEL),
                                      jnp.float32),
        mesh=mesh,
        scratch_types=[pltpu.VMEM((ROLE_VOCAB, D_MODEL), jnp.float32)],
        compiler_params=dataclasses.replace(pltpu.CompilerParams(),
                                            use_tc_tiling_on_sc=False,
                                            needs_layout_passes=False),
    )
    def sc_kernel(ct_hbm, rt_hbm, n_hbm, e_hbm, o_hbm, role_tile):
        pltpu.sync_copy(rt_hbm, role_tile)

        def node_body(i_vmem, o_vmem):
            pltpu.sync_copy(ct_hbm.at[i_vmem.at[0]], o_vmem)

        pltpu.emit_pipeline(
            node_body,
            grid=(N_NODES // W_NODE,),
            in_specs=[pl.BlockSpec((1, W_NODE), index_map=lambda i: (i, 0))],
            out_specs=[pl.BlockSpec((W_NODE, D_MODEL),
                                    index_map=lambda i: (i, 0))],
            core_axis_name=("core", "subcore"),
            dimension_semantics=(pltpu.PARALLEL,),
        )(n_hbm, o_hbm)

        def edge_body(i_vmem, o_vmem):
            lanes = jax.lax.iota(jnp.int32, 16)

            @pl.loop(0, W_EDGE // 16)
            def _(g):
                v16 = i_vmem[0, pl.ds(g * 16, 16)]
                rows = lanes + g * 16
                for d in range(D_MODEL):
                    d16 = jnp.full((16,), d, jnp.int32)
                    vals = plsc.load_gather(role_tile, [v16, d16])
                    plsc.store_scatter(o_vmem, [rows, d16], vals)

        pltpu.emit_pipeline(
            edge_body,
            grid=(N_EDGES // W_EDGE,),
            in_specs=[pl.BlockSpec((1, W_EDGE), index_map=lambda i: (i, 0))],
            out_specs=[pl.BlockSpec((W_EDGE, D_MODEL),
                                    index_map=lambda i: (i + N_NODES // W_EDGE, 0))],
            core_axis_name=("core", "subcore"),
            dimension_semantics=(pltpu.PARALLEL,),
        )(e_hbm, o_hbm)

    return sc_kernel(concept_table, role_table, nodes2d, edges2d)


# speedup vs baseline: 2.5806x; 2.5806x over previous
import dataclasses

import jax
import jax.numpy as jnp
from jax.experimental import pallas as pl
from jax.experimental.pallas import tpu as pltpu
from jax.experimental.pallas import tpu_sc as plsc

N_NODES = 100000
N_EDGES = 1600000
D_MODEL = 32

W_NODE = 400
W_EDGE = 800

TOTAL = N_NODES + N_EDGES
ROWS128 = TOTAL * D_MODEL // 128
TC_C = 512


def _tc_transpose_body(x_ref, o_ref):
    x = x_ref[...]
    o_ref[...] = x.reshape(TC_C, D_MODEL).T


def _tc_transpose(tmp2):
    grid = (TOTAL + TC_C - 1) // TC_C
    return pl.pallas_call(
        _tc_transpose_body,
        out_shape=jax.ShapeDtypeStruct((D_MODEL, TOTAL), jnp.float32),
        grid=(grid,),
        in_specs=[pl.BlockSpec((TC_C * D_MODEL // 128, 128),
                               lambda j: (j, 0))],
        out_specs=pl.BlockSpec((D_MODEL, TC_C), lambda j: (0, j)),
    )(tmp2)


def kernel(nodes, edges, edge_index, concept_table, role_table):
    del edge_index

    nodes2d = nodes.reshape(N_NODES // W_NODE, W_NODE).astype(jnp.int32)
    edges2d = edges.reshape(N_EDGES // W_EDGE, W_EDGE).astype(jnp.int32)

    mesh = plsc.VectorSubcoreMesh(core_axis_name="core",
                                  subcore_axis_name="subcore")

    @pl.kernel(
        out_type=jax.ShapeDtypeStruct((N_NODES + N_EDGES, 128),
                                      jnp.float32),
        mesh=mesh,
        compiler_params=dataclasses.replace(pltpu.CompilerParams(),
                                            use_tc_tiling_on_sc=False),
    )
    def sc_kernel(ct_hbm, rt_hbm, n_hbm, e_hbm, o_hbm):
        def node_body(i_vmem, o_vmem):
            pltpu.sync_copy(ct_hbm.at[i_vmem.at[0]], o_vmem)

        pltpu.emit_pipeline(
            node_body,
            grid=(N_NODES // W_NODE,),
            in_specs=[pl.BlockSpec((1, W_NODE), index_map=lambda i: (i, 0))],
            out_specs=[pl.BlockSpec((W_NODE, D_MODEL),
                                    index_map=lambda i: (i, 0))],
            core_axis_name=("core", "subcore"),
            dimension_semantics=(pltpu.PARALLEL,),
        )(n_hbm, o_hbm)

        def edge_body(i_vmem, o_vmem):
            pltpu.sync_copy(rt_hbm.at[i_vmem.at[0]], o_vmem)

        pltpu.emit_pipeline(
            edge_body,
            grid=(N_EDGES // W_EDGE,),
            in_specs=[pl.BlockSpec((1, W_EDGE), index_map=lambda i: (i, 0))],
            out_specs=[pl.BlockSpec((W_EDGE, D_MODEL),
                                    index_map=lambda i: (i + N_NODES // W_EDGE, 0))],
            core_axis_name=("core", "subcore"),
            dimension_semantics=(pltpu.PARALLEL,),
        )(e_hbm, o_hbm)

    tmp = sc_kernel(concept_table, role_table, nodes2d, edges2d)
    return tmp[:, :D_MODEL]


# speedup vs baseline: 2.6059x; 1.0098x over previous
import dataclasses

import jax
import jax.numpy as jnp
from jax.experimental import pallas as pl
from jax.experimental.pallas import tpu as pltpu
from jax.experimental.pallas import tpu_sc as plsc

N_NODES = 100000
N_EDGES = 1600000
D_MODEL = 32

W_NODE = 400
W_EDGE = 1000

TOTAL = N_NODES + N_EDGES
ROWS128 = TOTAL * D_MODEL // 128
TC_C = 512


def _tc_transpose_body(x_ref, o_ref):
    x = x_ref[...]
    o_ref[...] = x.reshape(TC_C, D_MODEL).T


def _tc_transpose(tmp2):
    grid = (TOTAL + TC_C - 1) // TC_C
    return pl.pallas_call(
        _tc_transpose_body,
        out_shape=jax.ShapeDtypeStruct((D_MODEL, TOTAL), jnp.float32),
        grid=(grid,),
        in_specs=[pl.BlockSpec((TC_C * D_MODEL // 128, 128),
                               lambda j: (j, 0))],
        out_specs=pl.BlockSpec((D_MODEL, TC_C), lambda j: (0, j)),
    )(tmp2)


def kernel(nodes, edges, edge_index, concept_table, role_table):
    del edge_index

    nodes2d = nodes.reshape(N_NODES // W_NODE, W_NODE).astype(jnp.int32)
    edges2d = edges.reshape(N_EDGES // W_EDGE, W_EDGE).astype(jnp.int32)

    mesh = plsc.VectorSubcoreMesh(core_axis_name="core",
                                  subcore_axis_name="subcore")

    @pl.kernel(
        out_type=jax.ShapeDtypeStruct((N_NODES + N_EDGES, 128),
                                      jnp.float32),
        mesh=mesh,
        compiler_params=dataclasses.replace(pltpu.CompilerParams(),
                                            use_tc_tiling_on_sc=False),
    )
    def sc_kernel(ct_hbm, rt_hbm, n_hbm, e_hbm, o_hbm):
        def node_body(i_vmem, o_vmem):
            pltpu.sync_copy(ct_hbm.at[i_vmem.at[0]], o_vmem)

        pltpu.emit_pipeline(
            node_body,
            grid=(N_NODES // W_NODE,),
            in_specs=[pl.BlockSpec((1, W_NODE), index_map=lambda i: (i, 0))],
            out_specs=[pl.BlockSpec((W_NODE, D_MODEL),
                                    index_map=lambda i: (i, 0))],
            core_axis_name=("core", "subcore"),
            dimension_semantics=(pltpu.PARALLEL,),
        )(n_hbm, o_hbm)

        def edge_body(i_vmem, o_vmem):
            pltpu.sync_copy(rt_hbm.at[i_vmem.at[0]], o_vmem)

        pltpu.emit_pipeline(
            edge_body,
            grid=(N_EDGES // W_EDGE,),
            in_specs=[pl.BlockSpec((1, W_EDGE), index_map=lambda i: (i, 0))],
            out_specs=[pl.BlockSpec((W_EDGE, D_MODEL),
                                    index_map=lambda i: (i + N_NODES // W_EDGE, 0))],
            core_axis_name=("core", "subcore"),
            dimension_semantics=(pltpu.PARALLEL,),
        )(e_hbm, o_hbm)

    tmp = sc_kernel(concept_table, role_table, nodes2d, edges2d)
    return tmp[:, :D_MODEL]


# speedup vs baseline: 2.6691x; 1.0242x over previous
import dataclasses

import jax
import jax.numpy as jnp
from jax.experimental import pallas as pl
from jax.experimental.pallas import tpu as pltpu
from jax.experimental.pallas import tpu_sc as plsc

N_NODES = 100000
N_EDGES = 1600000
D_MODEL = 32

W_NODE = 400
W_EDGE = 1000

_CPARAMS = dataclasses.replace(pltpu.CompilerParams(),
                               use_tc_tiling_on_sc=False)


def _mesh():
    return plsc.VectorSubcoreMesh(core_axis_name="core",
                                  subcore_axis_name="subcore")


def kernel(nodes, edges, edge_index, concept_table, role_table):
    del edge_index

    nodes2d = nodes.reshape(N_NODES // W_NODE, W_NODE).astype(jnp.int32)
    edges2d = edges.reshape(N_EDGES // W_EDGE, W_EDGE).astype(jnp.int32)

    @pl.kernel(
        out_type=jax.ShapeDtypeStruct((N_NODES + N_EDGES, 128), jnp.float32),
        mesh=_mesh(),
        compiler_params=_CPARAMS,
    )
    def edge_kernel(rt_hbm, e_hbm, o_hbm):
        def edge_body(i_vmem, o_vmem):
            pltpu.sync_copy(rt_hbm.at[i_vmem.at[0]], o_vmem)

        pltpu.emit_pipeline(
            edge_body,
            grid=(N_EDGES // W_EDGE,),
            in_specs=[pl.BlockSpec((1, W_EDGE), index_map=lambda i: (i, 0))],
            out_specs=[pl.BlockSpec((W_EDGE, D_MODEL),
                                    index_map=lambda i: (i + N_NODES // W_EDGE, 0))],
            core_axis_name=("core", "subcore"),
            dimension_semantics=(pltpu.PARALLEL,),
        )(e_hbm, o_hbm)

    @pl.kernel(
        out_type=jax.ShapeDtypeStruct((N_NODES, 128), jnp.float32),
        mesh=_mesh(),
        compiler_params=_CPARAMS,
    )
    def node_kernel(ct_hbm, n_hbm, o_hbm):
        def node_body(i_vmem, o_vmem):
            pltpu.sync_copy(ct_hbm.at[i_vmem.at[0]], o_vmem)

        pltpu.emit_pipeline(
            node_body,
            grid=(N_NODES // W_NODE,),
            in_specs=[pl.BlockSpec((1, W_NODE), index_map=lambda i: (i, 0))],
            out_specs=[pl.BlockSpec((W_NODE, D_MODEL),
                                    index_map=lambda i: (i, 0))],
            core_axis_name=("core", "subcore"),
            dimension_semantics=(pltpu.PARALLEL,),
        )(n_hbm, o_hbm)

    big = edge_kernel(role_table, edges2d)
    small = node_kernel(concept_table, nodes2d)
    return jax.lax.dynamic_update_slice(
        big[:, :D_MODEL], small[:, :D_MODEL], (0, 0))


# speedup vs baseline: 2.6725x; 1.0013x over previous
import dataclasses

import jax
import jax.numpy as jnp
from jax.experimental import pallas as pl
from jax.experimental.pallas import tpu as pltpu
from jax.experimental.pallas import tpu_sc as plsc

N_NODES = 100000
N_EDGES = 1600000
D_MODEL = 32

W_NODE = 1000
W_EDGE = 1000

_CPARAMS = dataclasses.replace(pltpu.CompilerParams(),
                               use_tc_tiling_on_sc=False)


def _mesh():
    return plsc.VectorSubcoreMesh(core_axis_name="core",
                                  subcore_axis_name="subcore")


def kernel(nodes, edges, edge_index, concept_table, role_table):
    del edge_index

    nodes2d = nodes.reshape(N_NODES // W_NODE, W_NODE).astype(jnp.int32)
    edges2d = edges.reshape(N_EDGES // W_EDGE, W_EDGE).astype(jnp.int32)

    @pl.kernel(
        out_type=jax.ShapeDtypeStruct((N_NODES + N_EDGES, 128), jnp.float32),
        mesh=_mesh(),
        compiler_params=_CPARAMS,
    )
    def edge_kernel(rt_hbm, e_hbm, o_hbm):
        def edge_body(i_vmem, o_vmem):
            pltpu.sync_copy(rt_hbm.at[i_vmem.at[0]], o_vmem)

        pltpu.emit_pipeline(
            edge_body,
            grid=(N_EDGES // W_EDGE,),
            in_specs=[pl.BlockSpec((1, W_EDGE), index_map=lambda i: (i, 0))],
            out_specs=[pl.BlockSpec((W_EDGE, D_MODEL),
                                    index_map=lambda i: (i + N_NODES // W_EDGE, 0))],
            core_axis_name=("core", "subcore"),
            dimension_semantics=(pltpu.PARALLEL,),
        )(e_hbm, o_hbm)

    @pl.kernel(
        out_type=jax.ShapeDtypeStruct((N_NODES, 128), jnp.float32),
        mesh=_mesh(),
        compiler_params=_CPARAMS,
    )
    def node_kernel(ct_hbm, n_hbm, o_hbm):
        def node_body(i_vmem, o_vmem):
            pltpu.sync_copy(ct_hbm.at[i_vmem.at[0]], o_vmem)

        pltpu.emit_pipeline(
            node_body,
            grid=(N_NODES // W_NODE,),
            in_specs=[pl.BlockSpec((1, W_NODE), index_map=lambda i: (i, 0))],
            out_specs=[pl.BlockSpec((W_NODE, D_MODEL),
                                    index_map=lambda i: (i, 0))],
            core_axis_name=("core", "subcore"),
            dimension_semantics=(pltpu.PARALLEL,),
        )(n_hbm, o_hbm)

    big = edge_kernel(role_table, edges2d)
    small = node_kernel(concept_table, nodes2d)
    return jax.lax.dynamic_update_slice(
        big[:, :D_MODEL], small[:, :D_MODEL], (0, 0))


# speedup vs baseline: 3.3350x; 1.2479x over previous
import dataclasses

import jax
import jax.numpy as jnp
from jax.experimental import pallas as pl
from jax.experimental.pallas import tpu as pltpu
from jax.experimental.pallas import tpu_sc as plsc

N_NODES = 100000
N_EDGES = 1600000
D_MODEL = 32

W_NODE = 1000
W_EDGE = 1000
ROLE_VOCAB = 1000
REPLICAS = 16

_CPARAMS = dataclasses.replace(pltpu.CompilerParams(),
                               use_tc_tiling_on_sc=False)


def _mesh():
    return plsc.VectorSubcoreMesh(core_axis_name="core",
                                  subcore_axis_name="subcore")


def kernel(nodes, edges, edge_index, concept_table, role_table):
    del edge_index

    nodes2d = nodes.reshape(N_NODES // W_NODE, W_NODE).astype(jnp.int32)
    edges2d = edges.reshape(N_EDGES // W_EDGE, W_EDGE).astype(jnp.int32)

    role_rep = jnp.tile(role_table, (REPLICAS, 1))
    rep_off = (jnp.arange(N_EDGES // W_EDGE, dtype=jnp.int32)[:, None]
               % REPLICAS) * ROLE_VOCAB
    edges2d = edges2d + rep_off

    @pl.kernel(
        out_type=jax.ShapeDtypeStruct((N_NODES + N_EDGES, 128), jnp.float32),
        mesh=_mesh(),
        compiler_params=_CPARAMS,
    )
    def edge_kernel(rt_hbm, e_hbm, o_hbm):
        def edge_body(i_vmem, o_vmem):
            pltpu.sync_copy(rt_hbm.at[i_vmem.at[0]], o_vmem)

        pltpu.emit_pipeline(
            edge_body,
            grid=(N_EDGES // W_EDGE,),
            in_specs=[pl.BlockSpec((1, W_EDGE), index_map=lambda i: (i, 0))],
            out_specs=[pl.BlockSpec((W_EDGE, D_MODEL),
                                    index_map=lambda i: (i + N_NODES // W_EDGE, 0))],
            core_axis_name=("core", "subcore"),
            dimension_semantics=(pltpu.PARALLEL,),
        )(e_hbm, o_hbm)

    @pl.kernel(
        out_type=jax.ShapeDtypeStruct((N_NODES, 128), jnp.float32),
        mesh=_mesh(),
        compiler_params=_CPARAMS,
    )
    def node_kernel(ct_hbm, n_hbm, o_hbm):
        def node_body(i_vmem, o_vmem):
            pltpu.sync_copy(ct_hbm.at[i_vmem.at[0]], o_vmem)

        pltpu.emit_pipeline(
            node_body,
            grid=(N_NODES // W_NODE,),
            in_specs=[pl.BlockSpec((1, W_NODE), index_map=lambda i: (i, 0))],
            out_specs=[pl.BlockSpec((W_NODE, D_MODEL),
                                    index_map=lambda i: (i, 0))],
            core_axis_name=("core", "subcore"),
            dimension_semantics=(pltpu.PARALLEL,),
        )(n_hbm, o_hbm)

    big = edge_kernel(role_rep, edges2d)
    small = node_kernel(concept_table, nodes2d)
    return jax.lax.dynamic_update_slice(
        big[:, :D_MODEL], small[:, :D_MODEL], (0, 0))


# speedup vs baseline: 3.3628x; 1.0083x over previous
import dataclasses

import jax
import jax.numpy as jnp
from jax.experimental import pallas as pl
from jax.experimental.pallas import tpu as pltpu
from jax.experimental.pallas import tpu_sc as plsc

N_NODES = 100000
N_EDGES = 1600000
D_MODEL = 32

W_NODE = 1000
W_EDGE = 1000
ROLE_VOCAB = 1000
REPLICAS = 32

_CPARAMS = dataclasses.replace(pltpu.CompilerParams(),
                               use_tc_tiling_on_sc=False)


def _mesh():
    return plsc.VectorSubcoreMesh(core_axis_name="core",
                                  subcore_axis_name="subcore")


def kernel(nodes, edges, edge_index, concept_table, role_table):
    del edge_index

    nodes2d = nodes.reshape(N_NODES // W_NODE, W_NODE).astype(jnp.int32)
    edges2d = edges.reshape(N_EDGES // W_EDGE, W_EDGE).astype(jnp.int32)

    role_rep = jnp.tile(role_table, (REPLICAS, 1))
    rep_off = (jnp.arange(W_EDGE, dtype=jnp.int32)[None, :]
               % REPLICAS) * ROLE_VOCAB
    edges2d = edges2d + rep_off

    @pl.kernel(
        out_type=jax.ShapeDtypeStruct((N_NODES + N_EDGES, 128), jnp.float32),
        mesh=_mesh(),
        compiler_params=_CPARAMS,
    )
    def edge_kernel(rt_hbm, e_hbm, o_hbm):
        def edge_body(i_vmem, o_vmem):
            pltpu.sync_copy(rt_hbm.at[i_vmem.at[0]], o_vmem)

        pltpu.emit_pipeline(
            edge_body,
            grid=(N_EDGES // W_EDGE,),
            in_specs=[pl.BlockSpec((1, W_EDGE), index_map=lambda i: (i, 0))],
            out_specs=[pl.BlockSpec((W_EDGE, D_MODEL),
                                    index_map=lambda i: (i + N_NODES // W_EDGE, 0))],
            core_axis_name=("core", "subcore"),
            dimension_semantics=(pltpu.PARALLEL,),
        )(e_hbm, o_hbm)

    @pl.kernel(
        out_type=jax.ShapeDtypeStruct((N_NODES, 128), jnp.float32),
        mesh=_mesh(),
        compiler_params=_CPARAMS,
    )
    def node_kernel(ct_hbm, n_hbm, o_hbm):
        def node_body(i_vmem, o_vmem):
            pltpu.sync_copy(ct_hbm.at[i_vmem.at[0]], o_vmem)

        pltpu.emit_pipeline(
            node_body,
            grid=(N_NODES // W_NODE,),
            in_specs=[pl.BlockSpec((1, W_NODE), index_map=lambda i: (i, 0))],
            out_specs=[pl.BlockSpec((W_NODE, D_MODEL),
                                    index_map=lambda i: (i, 0))],
            core_axis_name=("core", "subcore"),
            dimension_semantics=(pltpu.PARALLEL,),
        )(n_hbm, o_hbm)

    big = edge_kernel(role_rep, edges2d)
    small = node_kernel(concept_table, nodes2d)
    return jax.lax.dynamic_update_slice(
        big[:, :D_MODEL], small[:, :D_MODEL], (0, 0))
